# Initial kernel scaffold; baseline (speedup 1.0000x reference)
import jax
import jax.numpy as jnp
from jax.experimental import pallas as pl


def kernel(node_ids, edge_index, edge_weight, graph_ids, y_data, word_embeds, W1, b1, W2, b2, W_out, b_out):
    raise NotImplementedError("write your pallas kernel here")


# speedup vs baseline: 3.7565x; 3.7565x over previous
import functools
import jax
import jax.numpy as jnp
from jax import lax
from jax.experimental import pallas as pl
from jax.experimental.pallas import tpu as pltpu
from jax.experimental.pallas import tpu_sc as plsc

N = 10000
E = 320000
D = 128
B = 8
VOCAB = 50000

NC = 2
NS = 16
NW = NC * NS

C = 80
EDGES_PER_TILE = E // NW
NCHUNK = EDGES_PER_TILE // C
GCHUNKS = N // C

_mesh = plsc.VectorSubcoreMesh(core_axis_name="c", subcore_axis_name="s")


@functools.partial(
    pl.kernel,
    out_type=jax.ShapeDtypeStruct((N, D), jnp.float32),
    mesh=_mesh,
    scratch_types=[
        pltpu.VMEM((C,), jnp.int32),
        pltpu.VMEM((C, D), jnp.float32),
        pltpu.SemaphoreType.DMA,
    ],
)
def _emb_gather(table, ids, out, idx_v, rows_v, sem):
    wid = lax.axis_index("c") * NS + lax.axis_index("s")
    nk = (GCHUNKS - wid + NW - 1) // NW

    def body(i, _):
        base = (wid + i * NW) * C
        pltpu.sync_copy(ids.at[pl.ds(base, C)], idx_v)
        pltpu.async_copy(table.at[idx_v], rows_v, sem).wait()
        pltpu.sync_copy(rows_v, out.at[pl.ds(base, C)])
        return 0

    lax.fori_loop(0, nk, body, 0)


@functools.partial(
    pl.kernel,
    out_type=jax.ShapeDtypeStruct((NC * N, D), jnp.float32),
    mesh=_mesh,
    scratch_types=[
        pltpu.VMEM_SHARED((N, D), jnp.float32),
        pltpu.VMEM((C,), jnp.int32),
        pltpu.VMEM((C, D), jnp.float32),
        pltpu.VMEM((C, D), jnp.float32),
        pltpu.SemaphoreType.DMA,
        pltpu.SemaphoreType.DMA,
    ],
)
def _edge_counts(dst, cnt_out, cnt_sp, idx_d, zero_v, ones_v, sem_i, sem_s):
    cid = lax.axis_index("c")
    sid = lax.axis_index("s")
    wid = cid * NS + sid
    nrk = (GCHUNKS - sid + NS - 1) // NS

    zeros16 = jnp.zeros((16,), jnp.float32)
    ones16 = jnp.ones((16,), jnp.float32)
    for r in range(C):
        for j in range(D // 16):
            zero_v[r, pl.ds(j * 16, 16)] = zeros16
            ones_v[r, pl.ds(j * 16, 16)] = ones16

    def zbody(i, _):
        pltpu.sync_copy(zero_v, cnt_sp.at[pl.ds((sid + i * NS) * C, C)])
        return 0

    lax.fori_loop(0, nrk, zbody, 0)
    plsc.subcore_barrier()

    ebase = wid * EDGES_PER_TILE

    def chunk(k, _):
        pltpu.async_copy(dst.at[pl.ds(ebase + k * C, C)], idx_d, sem_i).wait()
        pltpu.async_copy(ones_v, cnt_sp.at[idx_d], sem_s, add=True).wait()
        return 0

    lax.fori_loop(0, NCHUNK, chunk, 0)
    plsc.subcore_barrier()

    def wbody(i, _):
        off = (sid + i * NS) * C
        pltpu.sync_copy(cnt_sp.at[pl.ds(off, C)], zero_v)
        pltpu.sync_copy(zero_v, cnt_out.at[pl.ds(cid * N + off, C)])
        return 0

    lax.fori_loop(0, nrk, wbody, 0)


@functools.partial(
    pl.kernel,
    out_type=jax.ShapeDtypeStruct((NC * N, D), jnp.float32),
    mesh=_mesh,
    scratch_types=[
        pltpu.VMEM_SHARED((N, D), jnp.float32),
        pltpu.VMEM((C,), jnp.int32),
        pltpu.VMEM((C,), jnp.int32),
        pltpu.VMEM((C,), jnp.float32),
        pltpu.VMEM((C, D), jnp.float32),
        pltpu.SemaphoreType.DMA,
        pltpu.SemaphoreType.DMA,
        pltpu.SemaphoreType.DMA,
    ],
)
def _msgpass(wh, src, dst, ew, s_out, acc, idx_s, idx_d, w_v, rows_v,
             sem, sem_i, sem_s):
    cid = lax.axis_index("c")
    sid = lax.axis_index("s")
    wid = cid * NS + sid
    nrk = (GCHUNKS - sid + NS - 1) // NS

    zeros16 = jnp.zeros((16,), jnp.float32)
    for r in range(C):
        for j in range(D // 16):
            rows_v[r, pl.ds(j * 16, 16)] = zeros16

    def zbody(i, _):
        pltpu.sync_copy(rows_v, acc.at[pl.ds((sid + i * NS) * C, C)])
        return 0

    lax.fori_loop(0, nrk, zbody, 0)
    plsc.subcore_barrier()

    ebase = wid * EDGES_PER_TILE

    def chunk(k, _):
        base = ebase + k * C
        pltpu.async_copy(src.at[pl.ds(base, C)], idx_s, sem_i).wait()
        pltpu.async_copy(dst.at[pl.ds(base, C)], idx_d, sem_i).wait()
        pltpu.async_copy(ew.at[pl.ds(base, C)], w_v, sem_i).wait()
        pltpu.async_copy(wh.at[idx_s], rows_v, sem).wait()

        def scale(g, _):
            w16 = w_v[pl.ds(g * 16, 16)]
            for l in range(16):
                e = g * 16 + l
                wgt = jnp.full((16,), w16[l], dtype=jnp.float32)
                for j in range(D // 16):
                    sl = pl.ds(j * 16, 16)
                    rows_v[e, sl] = rows_v[e, sl] * wgt
            return 0

        lax.fori_loop(0, C // 16, scale, 0)
        pltpu.async_copy(rows_v, acc.at[idx_d], sem_s, add=True).wait()
        return 0

    lax.fori_loop(0, NCHUNK, chunk, 0)
    plsc.subcore_barrier()

    def wbody(i, _):
        off = (sid + i * NS) * C
        pltpu.sync_copy(acc.at[pl.ds(off, C)], rows_v)
        pltpu.sync_copy(rows_v, s_out.at[pl.ds(cid * N + off, C)])
        return 0

    lax.fori_loop(0, nrk, wbody, 0)


_BLK = 1000
_GRID = N // _BLK


def _mm_body(x_ref, w_ref, b_ref, o_ref):
    o_ref[...] = (
        jnp.dot(x_ref[...], w_ref[...], preferred_element_type=jnp.float32) + b_ref[...]
    )


def _matmul_bias(x, W, b):
    return pl.pallas_call(
        _mm_body,
        grid=(_GRID,),
        in_specs=[
            pl.BlockSpec((_BLK, D), lambda i: (i, 0)),
            pl.BlockSpec((D, D), lambda i: (0, 0)),
            pl.BlockSpec((1, D), lambda i: (0, 0)),
        ],
        out_specs=pl.BlockSpec((_BLK, D), lambda i: (i, 0)),
        out_shape=jax.ShapeDtypeStruct((N, D), jnp.float32),
    )(x, W, b.reshape(1, D))


def _combine_mm_body(sa_ref, sb_ref, ca_ref, cb_ref, w_ref, b_ref, o_ref):
    cnt = ca_ref[...][:, 0:1] + cb_ref[...][:, 0:1]
    h = (sa_ref[...] + sb_ref[...]) / jnp.maximum(cnt, 1.0)
    h = jnp.where(h >= 0.0, h, 0.01 * h)
    o_ref[...] = (
        jnp.dot(h, w_ref[...], preferred_element_type=jnp.float32) + b_ref[...]
    )


def _combine_matmul(sa, sb, ca, cb, W, b):
    return pl.pallas_call(
        _combine_mm_body,
        grid=(_GRID,),
        in_specs=[
            pl.BlockSpec((_BLK, D), lambda i: (i, 0)),
            pl.BlockSpec((_BLK, D), lambda i: (i, 0)),
            pl.BlockSpec((_BLK, D), lambda i: (i, 0)),
            pl.BlockSpec((_BLK, D), lambda i: (i, 0)),
            pl.BlockSpec((D, D), lambda i: (0, 0)),
            pl.BlockSpec((1, D), lambda i: (0, 0)),
        ],
        out_specs=pl.BlockSpec((_BLK, D), lambda i: (i, 0)),
        out_shape=jax.ShapeDtypeStruct((N, D), jnp.float32),
    )(sa, sb, ca, cb, W, b.reshape(1, D))


def _head_body(sa_ref, sb_ref, ca_ref, cb_ref, gid_ref, y_ref, wo_ref, bo_ref,
               loss_ref, pred_ref, pooled, cntg):
    i = pl.program_id(0)
    cnt = ca_ref[...][:, 0:1] + cb_ref[...][:, 0:1]
    h = (sa_ref[...] + sb_ref[...]) / jnp.maximum(cnt, 1.0)
    gid = gid_ref[...]
    cols = lax.broadcasted_iota(jnp.int32, (1, B), 1)
    onehot = (gid == cols).astype(jnp.float32)
    part = lax.dot_general(onehot, h, (((0,), (0,)), ((), ())),
                           preferred_element_type=jnp.float32)
    pcnt = jnp.sum(onehot, axis=0, keepdims=True)

    @pl.when(i == 0)
    def _():
        pooled[...] = part
        cntg[...] = pcnt

    @pl.when(i > 0)
    def _():
        pooled[...] = pooled[...] + part
        cntg[...] = cntg[...] + pcnt

    @pl.when(i == _GRID - 1)
    def _():
        counts = cntg[...]
        max_len = jnp.maximum(jnp.max(counts), 1.0)
        wp = pooled[...] / max_len
        logits = (
            jnp.dot(wp, wo_ref[...], preferred_element_type=jnp.float32) + bo_ref[...]
        )
        z = logits
        y = y_ref[...]
        terms = jnp.maximum(z, 0.0) - z * y + jnp.log(1.0 + jnp.exp(-jnp.abs(z)))
        loss_ref[...] = jnp.sum(terms, keepdims=True).reshape(1, 1) / B
        pred_ref[...] = 1.0 / (1.0 + jnp.exp(-z))


def _head(sa, sb, ca, cb, gid, y, W_out, b_out):
    return pl.pallas_call(
        _head_body,
        grid=(_GRID,),
        in_specs=[
            pl.BlockSpec((_BLK, D), lambda i: (i, 0)),
            pl.BlockSpec((_BLK, D), lambda i: (i, 0)),
            pl.BlockSpec((_BLK, D), lambda i: (i, 0)),
            pl.BlockSpec((_BLK, D), lambda i: (i, 0)),
            pl.BlockSpec((_BLK, 1), lambda i: (i, 0)),
            pl.BlockSpec((B, 1), lambda i: (0, 0)),
            pl.BlockSpec((D, 1), lambda i: (0, 0)),
            pl.BlockSpec((1, 1), lambda i: (0, 0)),
        ],
        out_specs=[
            pl.BlockSpec((1, 1), lambda i: (0, 0)),
            pl.BlockSpec((B, 1), lambda i: (0, 0)),
        ],
        out_shape=[
            jax.ShapeDtypeStruct((1, 1), jnp.float32),
            jax.ShapeDtypeStruct((B, 1), jnp.float32),
        ],
        scratch_shapes=[
            pltpu.VMEM((B, D), jnp.float32),
            pltpu.VMEM((1, B), jnp.float32),
        ],
    )(sa, sb, ca, cb, gid, y, W_out, b_out.reshape(1, 1))


def kernel(node_ids, edge_index, edge_weight, graph_ids, y_data,
           word_embeds, W1, b1, W2, b2, W_out, b_out):
    node_ids = node_ids.astype(jnp.int32)
    src_i = edge_index[0].astype(jnp.int32)
    dst_i = edge_index[1].astype(jnp.int32)
    gid = graph_ids.astype(jnp.int32).reshape(N, 1)

    word_emb = _emb_gather(word_embeds, node_ids)
    cnt = _edge_counts(dst_i)
    wh1 = _matmul_bias(word_emb, W1, b1)
    s1 = _msgpass(wh1, src_i, dst_i, edge_weight)
    wh2 = _combine_matmul(s1[:N], s1[N:], cnt[:N], cnt[N:], W2, b2)
    s2 = _msgpass(wh2, src_i, dst_i, edge_weight)
    loss, y_pred = _head(s2[:N], s2[N:], cnt[:N], cnt[N:], gid,
                         y_data.reshape(B, 1), W_out, b_out)
    return (loss.reshape(()), y_pred)


# speedup vs baseline: 4.9403x; 1.3151x over previous
import functools
import jax
import jax.numpy as jnp
from jax import lax
from jax.experimental import pallas as pl
from jax.experimental.pallas import tpu as pltpu
from jax.experimental.pallas import tpu_sc as plsc

N = 10000
E = 320000
D = 128
B = 8
VOCAB = 50000

NC = 2
NS = 16
NW = NC * NS

C = 80
EDGES_PER_TILE = E // NW
NCHUNK = EDGES_PER_TILE // C
GCHUNKS = N // C

_mesh = plsc.VectorSubcoreMesh(core_axis_name="c", subcore_axis_name="s")


@functools.partial(
    pl.kernel,
    out_type=jax.ShapeDtypeStruct((N, D), jnp.float32),
    mesh=_mesh,
    scratch_types=[
        pltpu.VMEM((C,), jnp.int32),
        pltpu.VMEM((C, D), jnp.float32),
        pltpu.SemaphoreType.DMA,
    ],
)
def _emb_gather(table, ids, out, idx_v, rows_v, sem):
    wid = lax.axis_index("c") * NS + lax.axis_index("s")
    nk = (GCHUNKS - wid + NW - 1) // NW

    def body(i, _):
        base = (wid + i * NW) * C
        pltpu.sync_copy(ids.at[pl.ds(base, C)], idx_v)
        pltpu.async_copy(table.at[idx_v], rows_v, sem).wait()
        pltpu.sync_copy(rows_v, out.at[pl.ds(base, C)])
        return 0

    lax.fori_loop(0, nk, body, 0)


@functools.partial(
    pl.kernel,
    out_type=jax.ShapeDtypeStruct((NC * N, D), jnp.float32),
    mesh=_mesh,
    scratch_types=[
        pltpu.VMEM_SHARED((N, D), jnp.float32),
        pltpu.VMEM((NCHUNK, C), jnp.int32),
        pltpu.VMEM((C, D), jnp.float32),
        pltpu.VMEM((C, D), jnp.float32),
        pltpu.SemaphoreType.DMA,
        pltpu.SemaphoreType.DMA,
    ],
)
def _edge_counts(dst, cnt_out, cnt_sp, idx_d, zero_v, ones_v, sem_i, sem_s):
    cid = lax.axis_index("c")
    sid = lax.axis_index("s")
    wid = cid * NS + sid
    nrk = (GCHUNKS - sid + NS - 1) // NS

    zeros16 = jnp.zeros((16,), jnp.float32)
    ones16 = jnp.ones((16,), jnp.float32)
    for r in range(C):
        for j in range(D // 16):
            zero_v[r, pl.ds(j * 16, 16)] = zeros16
            ones_v[r, pl.ds(j * 16, 16)] = ones16

    def zbody(i, _):
        pltpu.sync_copy(zero_v, cnt_sp.at[pl.ds((sid + i * NS) * C, C)])
        return 0

    lax.fori_loop(0, nrk, zbody, 0)
    plsc.subcore_barrier()

    pltpu.async_copy(dst.at[wid], idx_d, sem_i).wait()

    def chunk(k, _):
        pltpu.async_copy(ones_v, cnt_sp.at[idx_d.at[k]], sem_s, add=True).wait()
        return 0

    lax.fori_loop(0, NCHUNK, chunk, 0)
    plsc.subcore_barrier()

    def wbody(i, _):
        off = (sid + i * NS) * C
        pltpu.sync_copy(cnt_sp.at[pl.ds(off, C)], zero_v)
        pltpu.sync_copy(zero_v, cnt_out.at[pl.ds(cid * N + off, C)])
        return 0

    lax.fori_loop(0, nrk, wbody, 0)


@functools.partial(
    pl.kernel,
    out_type=jax.ShapeDtypeStruct((NC * N, D), jnp.float32),
    mesh=_mesh,
    scratch_types=[
        pltpu.VMEM_SHARED((N, D), jnp.float32),
        pltpu.VMEM((NCHUNK, C), jnp.int32),
        pltpu.VMEM((NCHUNK, C), jnp.int32),
        pltpu.VMEM((C,), jnp.float32),
        pltpu.VMEM((C, D), jnp.float32),
        pltpu.SemaphoreType.DMA,
        pltpu.SemaphoreType.DMA,
        pltpu.SemaphoreType.DMA,
    ],
)
def _msgpass(wh, src, dst, ew, s_out, acc, idx_s, idx_d, w_v, rows_v,
             sem, sem_i, sem_s):
    cid = lax.axis_index("c")
    sid = lax.axis_index("s")
    wid = cid * NS + sid
    nrk = (GCHUNKS - sid + NS - 1) // NS

    zeros16 = jnp.zeros((16,), jnp.float32)
    for r in range(C):
        for j in range(D // 16):
            rows_v[r, pl.ds(j * 16, 16)] = zeros16

    def zbody(i, _):
        pltpu.sync_copy(rows_v, acc.at[pl.ds((sid + i * NS) * C, C)])
        return 0

    lax.fori_loop(0, nrk, zbody, 0)
    plsc.subcore_barrier()

    pltpu.async_copy(src.at[wid], idx_s, sem_i).wait()
    pltpu.async_copy(dst.at[wid], idx_d, sem_i).wait()

    def chunk(k, _):
        pltpu.async_copy(ew.at[wid, k], w_v, sem_i).wait()
        pltpu.async_copy(wh.at[idx_s.at[k]], rows_v, sem).wait()

        def scale(g, _):
            w16 = w_v[pl.ds(g * 16, 16)]
            for l in range(16):
                e = g * 16 + l
                wgt = jnp.full((16,), w16[l], dtype=jnp.float32)
                for j in range(D // 16):
                    sl = pl.ds(j * 16, 16)
                    rows_v[e, sl] = rows_v[e, sl] * wgt
            return 0

        lax.fori_loop(0, C // 16, scale, 0)
        pltpu.async_copy(rows_v, acc.at[idx_d.at[k]], sem_s, add=True).wait()
        return 0

    lax.fori_loop(0, NCHUNK, chunk, 0)
    plsc.subcore_barrier()

    def wbody(i, _):
        off = (sid + i * NS) * C
        pltpu.sync_copy(acc.at[pl.ds(off, C)], rows_v)
        pltpu.sync_copy(rows_v, s_out.at[pl.ds(cid * N + off, C)])
        return 0

    lax.fori_loop(0, nrk, wbody, 0)


_BLK = 1000
_GRID = N // _BLK


def _mm_body(x_ref, w_ref, b_ref, o_ref):
    o_ref[...] = (
        jnp.dot(x_ref[...], w_ref[...], preferred_element_type=jnp.float32) + b_ref[...]
    )


def _matmul_bias(x, W, b):
    return pl.pallas_call(
        _mm_body,
        grid=(_GRID,),
        in_specs=[
            pl.BlockSpec((_BLK, D), lambda i: (i, 0)),
            pl.BlockSpec((D, D), lambda i: (0, 0)),
            pl.BlockSpec((1, D), lambda i: (0, 0)),
        ],
        out_specs=pl.BlockSpec((_BLK, D), lambda i: (i, 0)),
        out_shape=jax.ShapeDtypeStruct((N, D), jnp.float32),
    )(x, W, b.reshape(1, D))


def _combine_mm_body(sa_ref, sb_ref, ca_ref, cb_ref, w_ref, b_ref, o_ref):
    cnt = ca_ref[...][:, 0:1] + cb_ref[...][:, 0:1]
    h = (sa_ref[...] + sb_ref[...]) / jnp.maximum(cnt, 1.0)
    h = jnp.where(h >= 0.0, h, 0.01 * h)
    o_ref[...] = (
        jnp.dot(h, w_ref[...], preferred_element_type=jnp.float32) + b_ref[...]
    )


def _combine_matmul(sa, sb, ca, cb, W, b):
    return pl.pallas_call(
        _combine_mm_body,
        grid=(_GRID,),
        in_specs=[
            pl.BlockSpec((_BLK, D), lambda i: (i, 0)),
            pl.BlockSpec((_BLK, D), lambda i: (i, 0)),
            pl.BlockSpec((_BLK, D), lambda i: (i, 0)),
            pl.BlockSpec((_BLK, D), lambda i: (i, 0)),
            pl.BlockSpec((D, D), lambda i: (0, 0)),
            pl.BlockSpec((1, D), lambda i: (0, 0)),
        ],
        out_specs=pl.BlockSpec((_BLK, D), lambda i: (i, 0)),
        out_shape=jax.ShapeDtypeStruct((N, D), jnp.float32),
    )(sa, sb, ca, cb, W, b.reshape(1, D))


def _head_body(sa_ref, sb_ref, ca_ref, cb_ref, gid_ref, y_ref, wo_ref, bo_ref,
               loss_ref, pred_ref, pooled, cntg):
    i = pl.program_id(0)
    cnt = ca_ref[...][:, 0:1] + cb_ref[...][:, 0:1]
    h = (sa_ref[...] + sb_ref[...]) / jnp.maximum(cnt, 1.0)
    gid = gid_ref[...]
    cols = lax.broadcasted_iota(jnp.int32, (1, B), 1)
    onehot = (gid == cols).astype(jnp.float32)
    part = lax.dot_general(onehot, h, (((0,), (0,)), ((), ())),
                           preferred_element_type=jnp.float32)
    pcnt = jnp.sum(onehot, axis=0, keepdims=True)

    @pl.when(i == 0)
    def _():
        pooled[...] = part
        cntg[...] = pcnt

    @pl.when(i > 0)
    def _():
        pooled[...] = pooled[...] + part
        cntg[...] = cntg[...] + pcnt

    @pl.when(i == _GRID - 1)
    def _():
        counts = cntg[...]
        max_len = jnp.maximum(jnp.max(counts), 1.0)
        wp = pooled[...] / max_len
        logits = (
            jnp.dot(wp, wo_ref[...], preferred_element_type=jnp.float32) + bo_ref[...]
        )
        z = logits
        y = y_ref[...]
        terms = jnp.maximum(z, 0.0) - z * y + jnp.log(1.0 + jnp.exp(-jnp.abs(z)))
        loss_ref[...] = jnp.sum(terms, keepdims=True).reshape(1, 1) / B
        pred_ref[...] = 1.0 / (1.0 + jnp.exp(-z))


def _head(sa, sb, ca, cb, gid, y, W_out, b_out):
    return pl.pallas_call(
        _head_body,
        grid=(_GRID,),
        in_specs=[
            pl.BlockSpec((_BLK, D), lambda i: (i, 0)),
            pl.BlockSpec((_BLK, D), lambda i: (i, 0)),
            pl.BlockSpec((_BLK, D), lambda i: (i, 0)),
            pl.BlockSpec((_BLK, D), lambda i: (i, 0)),
            pl.BlockSpec((_BLK, 1), lambda i: (i, 0)),
            pl.BlockSpec((B, 1), lambda i: (0, 0)),
            pl.BlockSpec((D, 1), lambda i: (0, 0)),
            pl.BlockSpec((1, 1), lambda i: (0, 0)),
        ],
        out_specs=[
            pl.BlockSpec((1, 1), lambda i: (0, 0)),
            pl.BlockSpec((B, 1), lambda i: (0, 0)),
        ],
        out_shape=[
            jax.ShapeDtypeStruct((1, 1), jnp.float32),
            jax.ShapeDtypeStruct((B, 1), jnp.float32),
        ],
        scratch_shapes=[
            pltpu.VMEM((B, D), jnp.float32),
            pltpu.VMEM((1, B), jnp.float32),
        ],
    )(sa, sb, ca, cb, gid, y, W_out, b_out.reshape(1, 1))


def kernel(node_ids, edge_index, edge_weight, graph_ids, y_data,
           word_embeds, W1, b1, W2, b2, W_out, b_out):
    node_ids = node_ids.astype(jnp.int32)
    src_i = edge_index[0].astype(jnp.int32).reshape(NW, NCHUNK, C)
    dst_i = edge_index[1].astype(jnp.int32).reshape(NW, NCHUNK, C)
    ew2 = edge_weight.reshape(NW, NCHUNK, C)
    gid = graph_ids.astype(jnp.int32).reshape(N, 1)

    word_emb = _emb_gather(word_embeds, node_ids)
    cnt = _edge_counts(dst_i)
    wh1 = _matmul_bias(word_emb, W1, b1)
    s1 = _msgpass(wh1, src_i, dst_i, ew2)
    wh2 = _combine_matmul(s1[:N], s1[N:], cnt[:N], cnt[N:], W2, b2)
    s2 = _msgpass(wh2, src_i, dst_i, ew2)
    loss, y_pred = _head(s2[:N], s2[N:], cnt[:N], cnt[N:], gid,
                         y_data.reshape(B, 1), W_out, b_out)
    return (loss.reshape(()), y_pred)


# speedup vs baseline: 5.7749x; 1.1689x over previous
import functools
import jax
import jax.numpy as jnp
from jax import lax
from jax.experimental import pallas as pl
from jax.experimental.pallas import tpu as pltpu
from jax.experimental.pallas import tpu_sc as plsc

N = 10000
E = 320000
D = 128
B = 8
VOCAB = 50000

NC = 2
NS = 16
NW = NC * NS

C = 80
EDGES_PER_TILE = E // NW
NCHUNK = EDGES_PER_TILE // C
GCHUNKS = N // C

_mesh = plsc.VectorSubcoreMesh(core_axis_name="c", subcore_axis_name="s")


@functools.partial(
    pl.kernel,
    out_type=jax.ShapeDtypeStruct((N, D), jnp.float32),
    mesh=_mesh,
    scratch_types=[
        pltpu.VMEM((C,), jnp.int32),
        pltpu.VMEM((C, D), jnp.float32),
        pltpu.SemaphoreType.DMA,
    ],
)
def _emb_gather(table, ids, out, idx_v, rows_v, sem):
    wid = lax.axis_index("c") * NS + lax.axis_index("s")
    nk = (GCHUNKS - wid + NW - 1) // NW

    def body(i, _):
        base = (wid + i * NW) * C
        pltpu.sync_copy(ids.at[pl.ds(base, C)], idx_v)
        pltpu.async_copy(table.at[idx_v], rows_v, sem).wait()
        pltpu.sync_copy(rows_v, out.at[pl.ds(base, C)])
        return 0

    lax.fori_loop(0, nk, body, 0)


@functools.partial(
    pl.kernel,
    out_type=jax.ShapeDtypeStruct((NC * N, D), jnp.float32),
    mesh=_mesh,
    scratch_types=[
        pltpu.VMEM_SHARED((N, D), jnp.float32),
        pltpu.VMEM((NCHUNK, C), jnp.int32),
        pltpu.VMEM((C, D), jnp.float32),
        pltpu.VMEM((C, D), jnp.float32),
        pltpu.SemaphoreType.DMA,
        pltpu.SemaphoreType.DMA,
    ],
)
def _edge_counts(dst, cnt_out, cnt_sp, idx_d, zero_v, ones_v, sem_i, sem_s):
    cid = lax.axis_index("c")
    sid = lax.axis_index("s")
    wid = cid * NS + sid
    nrk = (GCHUNKS - sid + NS - 1) // NS

    zeros16 = jnp.zeros((16,), jnp.float32)
    ones16 = jnp.ones((16,), jnp.float32)
    for r in range(C):
        for j in range(D // 16):
            zero_v[r, pl.ds(j * 16, 16)] = zeros16
            ones_v[r, pl.ds(j * 16, 16)] = ones16

    def zbody(i, _):
        pltpu.sync_copy(zero_v, cnt_sp.at[pl.ds((sid + i * NS) * C, C)])
        return 0

    lax.fori_loop(0, nrk, zbody, 0)
    plsc.subcore_barrier()

    pltpu.async_copy(dst.at[wid], idx_d, sem_i).wait()

    def chunk(k, _):
        pltpu.async_copy(ones_v, cnt_sp.at[idx_d.at[k]], sem_s, add=True).wait()
        return 0

    lax.fori_loop(0, NCHUNK, chunk, 0)
    plsc.subcore_barrier()

    def wbody(i, _):
        off = (sid + i * NS) * C
        pltpu.sync_copy(cnt_sp.at[pl.ds(off, C)], zero_v)
        pltpu.sync_copy(zero_v, cnt_out.at[pl.ds(cid * N + off, C)])
        return 0

    lax.fori_loop(0, nrk, wbody, 0)


@functools.partial(
    pl.kernel,
    out_type=jax.ShapeDtypeStruct((NC * N, D), jnp.float32),
    mesh=_mesh,
    scratch_types=[
        pltpu.VMEM_SHARED((N, D), jnp.float32),
        pltpu.VMEM((NCHUNK, C), jnp.int32),
        pltpu.VMEM((NCHUNK, C), jnp.int32),
        pltpu.VMEM((C,), jnp.float32),
        pltpu.VMEM((C, D), jnp.float32),
        pltpu.SemaphoreType.DMA,
        pltpu.SemaphoreType.DMA,
        pltpu.SemaphoreType.DMA,
    ],
)
def _msgpass(wh, src, dst, ew, s_out, acc, idx_s, idx_d, w_v, rows_v,
             sem, sem_i, sem_s):
    cid = lax.axis_index("c")
    sid = lax.axis_index("s")
    wid = cid * NS + sid
    nrk = (GCHUNKS - sid + NS - 1) // NS

    zeros16 = jnp.zeros((16,), jnp.float32)
    for r in range(C):
        for j in range(D // 16):
            rows_v[r, pl.ds(j * 16, 16)] = zeros16

    def zbody(i, _):
        pltpu.sync_copy(rows_v, acc.at[pl.ds((sid + i * NS) * C, C)])
        return 0

    lax.fori_loop(0, nrk, zbody, 0)
    plsc.subcore_barrier()

    pltpu.async_copy(src.at[wid], idx_s, sem_i).wait()
    pltpu.async_copy(dst.at[wid], idx_d, sem_i).wait()

    def chunk(k, _):
        cp_w = pltpu.async_copy(ew.at[wid, k], w_v, sem_i)
        cp_r = pltpu.async_copy(wh.at[idx_s.at[k]], rows_v, sem)
        cp_w.wait()
        cp_r.wait()

        def scale(g, _):
            w16 = w_v[pl.ds(g * 16, 16)]
            for l in range(16):
                e = g * 16 + l
                wgt = jnp.full((16,), w16[l], dtype=jnp.float32)
                for j in range(D // 16):
                    sl = pl.ds(j * 16, 16)
                    rows_v[e, sl] = rows_v[e, sl] * wgt
            return 0

        lax.fori_loop(0, C // 16, scale, 0)
        pltpu.async_copy(rows_v, acc.at[idx_d.at[k]], sem_s, add=True).wait()
        return 0

    lax.fori_loop(0, NCHUNK, chunk, 0)
    plsc.subcore_barrier()

    def wbody(i, _):
        off = (sid + i * NS) * C
        pltpu.sync_copy(acc.at[pl.ds(off, C)], rows_v)
        pltpu.sync_copy(rows_v, s_out.at[pl.ds(cid * N + off, C)])
        return 0

    lax.fori_loop(0, nrk, wbody, 0)


_BLK = 1000
_GRID = N // _BLK


def _mm_body(x_ref, w_ref, b_ref, o_ref):
    o_ref[...] = (
        jnp.dot(x_ref[...], w_ref[...], preferred_element_type=jnp.float32) + b_ref[...]
    )


def _matmul_bias(x, W, b):
    return pl.pallas_call(
        _mm_body,
        grid=(_GRID,),
        in_specs=[
            pl.BlockSpec((_BLK, D), lambda i: (i, 0)),
            pl.BlockSpec((D, D), lambda i: (0, 0)),
            pl.BlockSpec((1, D), lambda i: (0, 0)),
        ],
        out_specs=pl.BlockSpec((_BLK, D), lambda i: (i, 0)),
        out_shape=jax.ShapeDtypeStruct((N, D), jnp.float32),
    )(x, W, b.reshape(1, D))


def _combine_mm_body(sa_ref, sb_ref, ca_ref, cb_ref, w_ref, b_ref, o_ref):
    cnt = ca_ref[...][:, 0:1] + cb_ref[...][:, 0:1]
    h = (sa_ref[...] + sb_ref[...]) / jnp.maximum(cnt, 1.0)
    h = jnp.where(h >= 0.0, h, 0.01 * h)
    o_ref[...] = (
        jnp.dot(h, w_ref[...], preferred_element_type=jnp.float32) + b_ref[...]
    )


def _combine_matmul(sa, sb, ca, cb, W, b):
    return pl.pallas_call(
        _combine_mm_body,
        grid=(_GRID,),
        in_specs=[
            pl.BlockSpec((_BLK, D), lambda i: (i, 0)),
            pl.BlockSpec((_BLK, D), lambda i: (i, 0)),
            pl.BlockSpec((_BLK, D), lambda i: (i, 0)),
            pl.BlockSpec((_BLK, D), lambda i: (i, 0)),
            pl.BlockSpec((D, D), lambda i: (0, 0)),
            pl.BlockSpec((1, D), lambda i: (0, 0)),
        ],
        out_specs=pl.BlockSpec((_BLK, D), lambda i: (i, 0)),
        out_shape=jax.ShapeDtypeStruct((N, D), jnp.float32),
    )(sa, sb, ca, cb, W, b.reshape(1, D))


def _head_body(sa_ref, sb_ref, ca_ref, cb_ref, gid_ref, y_ref, wo_ref, bo_ref,
               loss_ref, pred_ref, pooled, cntg):
    i = pl.program_id(0)
    cnt = ca_ref[...][:, 0:1] + cb_ref[...][:, 0:1]
    h = (sa_ref[...] + sb_ref[...]) / jnp.maximum(cnt, 1.0)
    gid = gid_ref[...]
    cols = lax.broadcasted_iota(jnp.int32, (1, B), 1)
    onehot = (gid == cols).astype(jnp.float32)
    part = lax.dot_general(onehot, h, (((0,), (0,)), ((), ())),
                           preferred_element_type=jnp.float32)
    pcnt = jnp.sum(onehot, axis=0, keepdims=True)

    @pl.when(i == 0)
    def _():
        pooled[...] = part
        cntg[...] = pcnt

    @pl.when(i > 0)
    def _():
        pooled[...] = pooled[...] + part
        cntg[...] = cntg[...] + pcnt

    @pl.when(i == _GRID - 1)
    def _():
        counts = cntg[...]
        max_len = jnp.maximum(jnp.max(counts), 1.0)
        wp = pooled[...] / max_len
        logits = (
            jnp.dot(wp, wo_ref[...], preferred_element_type=jnp.float32) + bo_ref[...]
        )
        z = logits
        y = y_ref[...]
        terms = jnp.maximum(z, 0.0) - z * y + jnp.log(1.0 + jnp.exp(-jnp.abs(z)))
        loss_ref[...] = jnp.sum(terms, keepdims=True).reshape(1, 1) / B
        pred_ref[...] = 1.0 / (1.0 + jnp.exp(-z))


def _head(sa, sb, ca, cb, gid, y, W_out, b_out):
    return pl.pallas_call(
        _head_body,
        grid=(_GRID,),
        in_specs=[
            pl.BlockSpec((_BLK, D), lambda i: (i, 0)),
            pl.BlockSpec((_BLK, D), lambda i: (i, 0)),
            pl.BlockSpec((_BLK, D), lambda i: (i, 0)),
            pl.BlockSpec((_BLK, D), lambda i: (i, 0)),
            pl.BlockSpec((_BLK, 1), lambda i: (i, 0)),
            pl.BlockSpec((B, 1), lambda i: (0, 0)),
            pl.BlockSpec((D, 1), lambda i: (0, 0)),
            pl.BlockSpec((1, 1), lambda i: (0, 0)),
        ],
        out_specs=[
            pl.BlockSpec((1, 1), lambda i: (0, 0)),
            pl.BlockSpec((B, 1), lambda i: (0, 0)),
        ],
        out_shape=[
            jax.ShapeDtypeStruct((1, 1), jnp.float32),
            jax.ShapeDtypeStruct((B, 1), jnp.float32),
        ],
        scratch_shapes=[
            pltpu.VMEM((B, D), jnp.float32),
            pltpu.VMEM((1, B), jnp.float32),
        ],
    )(sa, sb, ca, cb, gid, y, W_out, b_out.reshape(1, 1))


def kernel(node_ids, edge_index, edge_weight, graph_ids, y_data,
           word_embeds, W1, b1, W2, b2, W_out, b_out):
    node_ids = node_ids.astype(jnp.int32)
    src_i = edge_index[0].astype(jnp.int32).reshape(NW, NCHUNK, C)
    dst_i = edge_index[1].astype(jnp.int32).reshape(NW, NCHUNK, C)
    ew2 = edge_weight.reshape(NW, NCHUNK, C)
    gid = graph_ids.astype(jnp.int32).reshape(N, 1)

    word_emb = _emb_gather(word_embeds, node_ids)
    cnt = _edge_counts(dst_i)
    wh1 = _matmul_bias(word_emb, W1, b1)
    s1 = _msgpass(wh1, src_i, dst_i, ew2)
    wh2 = _combine_matmul(s1[:N], s1[N:], cnt[:N], cnt[N:], W2, b2)
    s2 = _msgpass(wh2, src_i, dst_i, ew2)
    loss, y_pred = _head(s2[:N], s2[N:], cnt[:N], cnt[N:], gid,
                         y_data.reshape(B, 1), W_out, b_out)
    return (loss.reshape(()), y_pred)
